# baseline (device time: 41068 ns/iter reference)
import jax
import jax.numpy as jnp
from jax import lax
from jax.experimental import pallas as pl
from jax.experimental.pallas import tpu as pltpu

N_DEV = 4
SUB = 4


def kernel(x, w_mat, scale_x, scale_w):
    m_tot, k_per = x.shape
    _, n = w_mat.shape
    m_per = m_tot // N_DEV
    m_sub = m_per // SUB

    def body(x_ref, w_ref, sx_ref, sw_ref, out_ref, xg_ref, send_sems, recv_sems):
        my = lax.axis_index("i")

        with jax.named_scope("barrier"):
            barrier_sem = pltpu.get_barrier_semaphore()
            for d in range(1, N_DEV):
                pl.semaphore_signal(
                    barrier_sem, inc=1,
                    device_id=((my + d) % N_DEV,),
                    device_id_type=pl.DeviceIdType.MESH,
                )
            pl.semaphore_wait(barrier_sem, N_DEV - 1)

        sends = []
        with jax.named_scope("send_issue"):
            for k in range(SUB):
                for d in (2, 1, 3):
                    peer = (my + d) % N_DEV
                    rdma = pltpu.make_async_remote_copy(
                        src_ref=x_ref.at[
                            pl.ds(peer * m_per + k * m_sub, m_sub), :],
                        dst_ref=xg_ref.at[my, pl.ds(k * m_sub, m_sub), :],
                        send_sem=send_sems.at[d - 1, k],
                        recv_sem=recv_sems.at[d - 1, k],
                        device_id=(peer,),
                        device_id_type=pl.DeviceIdType.MESH,
                    )
                    rdma.start()
                    sends.append(rdma)

        def partial(xs, k_slice):
            return lax.dot_general(
                xs.astype(jnp.bfloat16),
                w_ref[pl.ds(k_slice * k_per, k_per), :].astype(jnp.bfloat16),
                (((1,), (0,)), ((), ())),
                preferred_element_type=jnp.float32,
            )

        acc = []
        with jax.named_scope("gemm_local"):
            for k in range(SUB):
                acc.append(
                    partial(x_ref[pl.ds(my * m_per + k * m_sub, m_sub), :], my))

        s = sx_ref[0] * sw_ref[0]
        for k in range(SUB):
            for d in (1, 3, 2):
                src = (my + N_DEV - d) % N_DEV
                with jax.named_scope(f"wait_recv#r={k}_d={d}"):
                    recv = pltpu.make_async_remote_copy(
                        src_ref=x_ref.at[pl.ds(0, m_sub), :],
                        dst_ref=xg_ref.at[src, pl.ds(k * m_sub, m_sub), :],
                        send_sem=send_sems.at[d - 1, k],
                        recv_sem=recv_sems.at[d - 1, k],
                        device_id=(my,),
                        device_id_type=pl.DeviceIdType.MESH,
                    )
                    recv.wait_recv()
                with jax.named_scope(f"gemm#r={k}_d={d}"):
                    acc[k] = acc[k] + partial(
                        xg_ref[src, pl.ds(k * m_sub, m_sub), :], src)
            with jax.named_scope(f"epilogue#r={k}"):
                y = acc[k] * s
                out_ref[pl.ds(k * m_sub, m_sub), :] = y * jax.nn.sigmoid(y)

        with jax.named_scope("wait_send"):
            for rdma in sends:
                rdma.wait_send()

    return pl.pallas_call(
        body,
        out_shape=jax.ShapeDtypeStruct((m_per, n), jnp.float32),
        in_specs=[
            pl.BlockSpec(memory_space=pltpu.VMEM),
            pl.BlockSpec(memory_space=pltpu.VMEM),
            pl.BlockSpec(memory_space=pltpu.SMEM),
            pl.BlockSpec(memory_space=pltpu.SMEM),
        ],
        out_specs=pl.BlockSpec(memory_space=pltpu.VMEM),
        scratch_shapes=[
            pltpu.VMEM((N_DEV, m_per, k_per), jnp.int8),
            pltpu.SemaphoreType.DMA((N_DEV - 1, SUB)),
            pltpu.SemaphoreType.DMA((N_DEV - 1, SUB)),
        ],
        compiler_params=pltpu.CompilerParams(collective_id=0),
    )(x, w_mat, scale_x, scale_w)


# device time: 38297 ns/iter; 1.0724x vs baseline; 1.0724x over previous
import jax
import jax.numpy as jnp
from jax import lax
from jax.experimental import pallas as pl
from jax.experimental.pallas import tpu as pltpu

N_DEV = 4
SUB = 4


def kernel(x, w_mat, scale_x, scale_w):
    m_tot, k_per = x.shape
    _, n = w_mat.shape
    m_per = m_tot // N_DEV
    m_sub = m_per // SUB

    def body(x_ref, w_ref, sx_ref, sw_ref, out_ref, xg_ref, send_sems, recv_sems):
        my = lax.axis_index("i")

        with jax.named_scope("barrier"):
            barrier_sem = pltpu.get_barrier_semaphore()
            for d in range(1, N_DEV):
                pl.semaphore_signal(
                    barrier_sem, inc=1,
                    device_id=((my + d) % N_DEV,),
                    device_id_type=pl.DeviceIdType.MESH,
                )
            pl.semaphore_wait(barrier_sem, N_DEV - 1)

        sends = []
        with jax.named_scope("send_issue"):
            for k in range(SUB):
                for d in (1, 3, 2):
                    peer = (my + d) % N_DEV
                    rdma = pltpu.make_async_remote_copy(
                        src_ref=x_ref.at[
                            pl.ds(peer * m_per + k * m_sub, m_sub), :],
                        dst_ref=xg_ref.at[my, pl.ds(k * m_sub, m_sub), :],
                        send_sem=send_sems.at[d - 1, k],
                        recv_sem=recv_sems.at[d - 1, k],
                        device_id=(peer,),
                        device_id_type=pl.DeviceIdType.MESH,
                    )
                    rdma.start()
                    sends.append(rdma)

        def partial(xs, k_slice):
            return lax.dot_general(
                xs.astype(jnp.bfloat16),
                w_ref[pl.ds(k_slice * k_per, k_per), :].astype(jnp.bfloat16),
                (((1,), (0,)), ((), ())),
                preferred_element_type=jnp.float32,
            )

        acc = []
        with jax.named_scope("gemm_local"):
            for k in range(SUB):
                acc.append(
                    partial(x_ref[pl.ds(my * m_per + k * m_sub, m_sub), :], my))

        s = sx_ref[0] * sw_ref[0]
        for k in range(SUB):
            for d in (1, 3, 2):
                src = (my + N_DEV - d) % N_DEV
                with jax.named_scope(f"wait_recv#r={k}_d={d}"):
                    recv = pltpu.make_async_remote_copy(
                        src_ref=x_ref.at[pl.ds(0, m_sub), :],
                        dst_ref=xg_ref.at[src, pl.ds(k * m_sub, m_sub), :],
                        send_sem=send_sems.at[d - 1, k],
                        recv_sem=recv_sems.at[d - 1, k],
                        device_id=(my,),
                        device_id_type=pl.DeviceIdType.MESH,
                    )
                    recv.wait_recv()
                with jax.named_scope(f"gemm#r={k}_d={d}"):
                    acc[k] = acc[k] + partial(
                        xg_ref[src, pl.ds(k * m_sub, m_sub), :], src)
            with jax.named_scope(f"epilogue#r={k}"):
                y = acc[k] * s
                out_ref[pl.ds(k * m_sub, m_sub), :] = y * jax.nn.sigmoid(y)

        with jax.named_scope("wait_send"):
            for rdma in sends:
                rdma.wait_send()

    return pl.pallas_call(
        body,
        out_shape=jax.ShapeDtypeStruct((m_per, n), jnp.float32),
        in_specs=[
            pl.BlockSpec(memory_space=pltpu.VMEM),
            pl.BlockSpec(memory_space=pltpu.VMEM),
            pl.BlockSpec(memory_space=pltpu.SMEM),
            pl.BlockSpec(memory_space=pltpu.SMEM),
        ],
        out_specs=pl.BlockSpec(memory_space=pltpu.VMEM),
        scratch_shapes=[
            pltpu.VMEM((N_DEV, m_per, k_per), jnp.int8),
            pltpu.SemaphoreType.DMA((N_DEV - 1, SUB)),
            pltpu.SemaphoreType.DMA((N_DEV - 1, SUB)),
        ],
        compiler_params=pltpu.CompilerParams(collective_id=0),
    )(x, w_mat, scale_x, scale_w)


# device time: 34955 ns/iter; 1.1749x vs baseline; 1.0956x over previous
import jax
import jax.numpy as jnp
from jax import lax
from jax.experimental import pallas as pl
from jax.experimental.pallas import tpu as pltpu

N_DEV = 4
SUB = 4


def kernel(x, w_mat, scale_x, scale_w):
    m_tot, k_per = x.shape
    _, n = w_mat.shape
    m_per = m_tot // N_DEV
    m_sub = m_per // SUB

    def body(x_ref, w_ref, sx_ref, sw_ref, out_ref, xg_ref, wv_ref,
             send_sems, recv_sems, w_sems):
        my = lax.axis_index("i")

        with jax.named_scope("stage_w"):
            w_cps = []
            for i, off in enumerate((0, 3, 1, 2)):
                j = (my + off) % N_DEV
                cp = pltpu.make_async_copy(
                    w_ref.at[pl.ds(j * k_per, k_per), :],
                    wv_ref.at[j], w_sems.at[i])
                cp.start()
                w_cps.append(cp)

        with jax.named_scope("barrier"):
            barrier_sem = pltpu.get_barrier_semaphore()
            for d in range(1, N_DEV):
                pl.semaphore_signal(
                    barrier_sem, inc=1,
                    device_id=((my + d) % N_DEV,),
                    device_id_type=pl.DeviceIdType.MESH,
                )
            pl.semaphore_wait(barrier_sem, N_DEV - 1)

        sends = []
        with jax.named_scope("send_issue"):
            for k in range(SUB):
                for d in (1, 3, 2):
                    peer = (my + d) % N_DEV
                    rdma = pltpu.make_async_remote_copy(
                        src_ref=x_ref.at[
                            pl.ds(peer * m_per + k * m_sub, m_sub), :],
                        dst_ref=xg_ref.at[my, pl.ds(k * m_sub, m_sub), :],
                        send_sem=send_sems.at[d - 1, k],
                        recv_sem=recv_sems.at[d - 1, k],
                        device_id=(peer,),
                        device_id_type=pl.DeviceIdType.MESH,
                    )
                    rdma.start()
                    sends.append(rdma)

        def partial(xs, k_slice):
            return lax.dot_general(
                xs.astype(jnp.bfloat16),
                wv_ref[k_slice].astype(jnp.bfloat16),
                (((1,), (0,)), ((), ())),
                preferred_element_type=jnp.float32,
            )

        acc = []
        with jax.named_scope("gemm_local"):
            w_cps[0].wait()
            for k in range(SUB):
                acc.append(
                    partial(x_ref[pl.ds(my * m_per + k * m_sub, m_sub), :], my))

        s = sx_ref[0] * sw_ref[0]
        w_waited = {1: False, 3: False, 2: False}
        w_cp_for_d = {1: w_cps[1], 3: w_cps[2], 2: w_cps[3]}
        for k in range(SUB):
            for d in (1, 3, 2):
                src = (my + N_DEV - d) % N_DEV
                with jax.named_scope(f"wait_recv#r={k}_d={d}"):
                    recv = pltpu.make_async_remote_copy(
                        src_ref=x_ref.at[pl.ds(0, m_sub), :],
                        dst_ref=xg_ref.at[src, pl.ds(k * m_sub, m_sub), :],
                        send_sem=send_sems.at[d - 1, k],
                        recv_sem=recv_sems.at[d - 1, k],
                        device_id=(my,),
                        device_id_type=pl.DeviceIdType.MESH,
                    )
                    recv.wait_recv()
                    if not w_waited[d]:
                        w_cp_for_d[d].wait()
                        w_waited[d] = True
                with jax.named_scope(f"gemm#r={k}_d={d}"):
                    acc[k] = acc[k] + partial(
                        xg_ref[src, pl.ds(k * m_sub, m_sub), :], src)
            with jax.named_scope(f"epilogue#r={k}"):
                y = acc[k] * s
                out_ref[pl.ds(k * m_sub, m_sub), :] = y * jax.nn.sigmoid(y)

        with jax.named_scope("wait_send"):
            for rdma in sends:
                rdma.wait_send()

    w_mat = pltpu.with_memory_space_constraint(w_mat, pltpu.MemorySpace.HBM)
    return pl.pallas_call(
        body,
        out_shape=jax.ShapeDtypeStruct((m_per, n), jnp.float32),
        in_specs=[
            pl.BlockSpec(memory_space=pltpu.VMEM),
            pl.BlockSpec(memory_space=pltpu.MemorySpace.HBM),
            pl.BlockSpec(memory_space=pltpu.SMEM),
            pl.BlockSpec(memory_space=pltpu.SMEM),
        ],
        out_specs=pl.BlockSpec(memory_space=pltpu.VMEM),
        scratch_shapes=[
            pltpu.VMEM((N_DEV, m_per, k_per), jnp.int8),
            pltpu.VMEM((N_DEV, k_per, n), jnp.int8),
            pltpu.SemaphoreType.DMA((N_DEV - 1, SUB)),
            pltpu.SemaphoreType.DMA((N_DEV - 1, SUB)),
            pltpu.SemaphoreType.DMA((N_DEV,)),
        ],
        compiler_params=pltpu.CompilerParams(collective_id=0),
    )(x, w_mat, scale_x, scale_w)


# device time: 34131 ns/iter; 1.2032x vs baseline; 1.0241x over previous
import jax
import jax.numpy as jnp
from jax import lax
from jax.experimental import pallas as pl
from jax.experimental.pallas import tpu as pltpu

N_DEV = 4
SUB = 4


def kernel(x, w_mat, scale_x, scale_w):
    m_tot, k_per = x.shape
    _, n = w_mat.shape
    m_per = m_tot // N_DEV
    m_sub = m_per // SUB

    def body(x_ref, w_ref, sx_ref, sw_ref, out_ref, xg_ref, xv_ref, wv_ref,
             send_sems, recv_sems, x_sem, w_sems):
        my = lax.axis_index("i")

        with jax.named_scope("stage"):
            x_cp = pltpu.make_async_copy(x_ref, xv_ref, x_sem)
            x_cp.start()
            w_cps = []
            for i, off in enumerate((0, 3, 1, 2)):
                j = (my + off) % N_DEV
                cp = pltpu.make_async_copy(
                    w_ref.at[pl.ds(j * k_per, k_per), :],
                    wv_ref.at[j], w_sems.at[i])
                cp.start()
                w_cps.append(cp)

        with jax.named_scope("barrier"):
            barrier_sem = pltpu.get_barrier_semaphore()
            for d in range(1, N_DEV):
                pl.semaphore_signal(
                    barrier_sem, inc=1,
                    device_id=((my + d) % N_DEV,),
                    device_id_type=pl.DeviceIdType.MESH,
                )
            pl.semaphore_wait(barrier_sem, N_DEV - 1)

        sends = []
        with jax.named_scope("send_issue"):
            x_cp.wait()
            for k in range(SUB):
                for d in (1, 3, 2):
                    peer = (my + d) % N_DEV
                    rdma = pltpu.make_async_remote_copy(
                        src_ref=xv_ref.at[
                            pl.ds(peer * m_per + k * m_sub, m_sub), :],
                        dst_ref=xg_ref.at[my, pl.ds(k * m_sub, m_sub), :],
                        send_sem=send_sems.at[d - 1, k],
                        recv_sem=recv_sems.at[d - 1, k],
                        device_id=(peer,),
                        device_id_type=pl.DeviceIdType.MESH,
                    )
                    rdma.start()
                    sends.append(rdma)

        def partial(xs, k_slice):
            return lax.dot_general(
                xs.astype(jnp.bfloat16),
                wv_ref[k_slice].astype(jnp.bfloat16),
                (((1,), (0,)), ((), ())),
                preferred_element_type=jnp.float32,
            )

        acc = []
        with jax.named_scope("gemm_local"):
            w_cps[0].wait()
            for k in range(SUB):
                acc.append(
                    partial(xv_ref[pl.ds(my * m_per + k * m_sub, m_sub), :],
                            my))

        s = sx_ref[0] * sw_ref[0]
        w_waited = {1: False, 3: False, 2: False}
        w_cp_for_d = {1: w_cps[1], 3: w_cps[2], 2: w_cps[3]}
        for k in range(SUB):
            for d in (1, 3, 2):
                src = (my + N_DEV - d) % N_DEV
                with jax.named_scope(f"wait_recv#r={k}_d={d}"):
                    recv = pltpu.make_async_remote_copy(
                        src_ref=xv_ref.at[pl.ds(0, m_sub), :],
                        dst_ref=xg_ref.at[src, pl.ds(k * m_sub, m_sub), :],
                        send_sem=send_sems.at[d - 1, k],
                        recv_sem=recv_sems.at[d - 1, k],
                        device_id=(my,),
                        device_id_type=pl.DeviceIdType.MESH,
                    )
                    recv.wait_recv()
                    if not w_waited[d]:
                        w_cp_for_d[d].wait()
                        w_waited[d] = True
                with jax.named_scope(f"gemm#r={k}_d={d}"):
                    acc[k] = acc[k] + partial(
                        xg_ref[src, pl.ds(k * m_sub, m_sub), :], src)
            with jax.named_scope(f"epilogue#r={k}"):
                y = acc[k] * s
                out_ref[pl.ds(k * m_sub, m_sub), :] = y * jax.nn.sigmoid(y)

        with jax.named_scope("wait_send"):
            for rdma in sends:
                rdma.wait_send()

    x = pltpu.with_memory_space_constraint(x, pltpu.MemorySpace.HBM)
    w_mat = pltpu.with_memory_space_constraint(w_mat, pltpu.MemorySpace.HBM)
    return pl.pallas_call(
        body,
        out_shape=jax.ShapeDtypeStruct((m_per, n), jnp.float32),
        in_specs=[
            pl.BlockSpec(memory_space=pltpu.MemorySpace.HBM),
            pl.BlockSpec(memory_space=pltpu.MemorySpace.HBM),
            pl.BlockSpec(memory_space=pltpu.SMEM),
            pl.BlockSpec(memory_space=pltpu.SMEM),
        ],
        out_specs=pl.BlockSpec(memory_space=pltpu.VMEM),
        scratch_shapes=[
            pltpu.VMEM((N_DEV, m_per, k_per), jnp.int8),
            pltpu.VMEM((m_tot, k_per), jnp.int8),
            pltpu.VMEM((N_DEV, k_per, n), jnp.int8),
            pltpu.SemaphoreType.DMA((N_DEV - 1, SUB)),
            pltpu.SemaphoreType.DMA((N_DEV - 1, SUB)),
            pltpu.SemaphoreType.DMA,
            pltpu.SemaphoreType.DMA((N_DEV,)),
        ],
        compiler_params=pltpu.CompilerParams(collective_id=0),
    )(x, w_mat, scale_x, scale_w)


# device time: 33774 ns/iter; 1.2160x vs baseline; 1.0106x over previous
import jax
import jax.numpy as jnp
from jax import lax
from jax.experimental import pallas as pl
from jax.experimental.pallas import tpu as pltpu

N_DEV = 4
SUB = 4


def kernel(x, w_mat, scale_x, scale_w):
    m_tot, k_per = x.shape
    _, n = w_mat.shape
    m_per = m_tot // N_DEV
    m_sub = m_per // SUB

    def body(x_ref, w_ref, sx_ref, sw_ref, out_ref, xg_ref, xv_ref, wv_ref,
             send_sems, recv_sems, x_sem, w_sems):
        my = lax.axis_index("i")

        with jax.named_scope("stage"):
            x_cp = pltpu.make_async_copy(
                x_ref.at[pl.ds(my * m_per, m_per), :], xv_ref, x_sem)
            x_cp.start()
            w_cps = []
            for i, off in enumerate((0, 3, 1, 2)):
                j = (my + off) % N_DEV
                cp = pltpu.make_async_copy(
                    w_ref.at[pl.ds(j * k_per, k_per), :],
                    wv_ref.at[j], w_sems.at[i])
                cp.start()
                w_cps.append(cp)

        with jax.named_scope("barrier"):
            barrier_sem = pltpu.get_barrier_semaphore()
            for d in range(1, N_DEV):
                pl.semaphore_signal(
                    barrier_sem, inc=1,
                    device_id=((my + d) % N_DEV,),
                    device_id_type=pl.DeviceIdType.MESH,
                )
            pl.semaphore_wait(barrier_sem, N_DEV - 1)

        sends = []
        with jax.named_scope("send_issue"):
            for k in range(SUB):
                for d in (1, 3, 2):
                    peer = (my + d) % N_DEV
                    rdma = pltpu.make_async_remote_copy(
                        src_ref=x_ref.at[
                            pl.ds(peer * m_per + k * m_sub, m_sub), :],
                        dst_ref=xg_ref.at[my, pl.ds(k * m_sub, m_sub), :],
                        send_sem=send_sems.at[d - 1, k],
                        recv_sem=recv_sems.at[d - 1, k],
                        device_id=(peer,),
                        device_id_type=pl.DeviceIdType.MESH,
                    )
                    rdma.start()
                    sends.append(rdma)

        def partial(xs, k_slice):
            return lax.dot_general(
                xs.astype(jnp.bfloat16),
                wv_ref[k_slice].astype(jnp.bfloat16),
                (((1,), (0,)), ((), ())),
                preferred_element_type=jnp.float32,
            )

        acc = []
        with jax.named_scope("gemm_local"):
            x_cp.wait()
            w_cps[0].wait()
            for k in range(SUB):
                acc.append(
                    partial(xv_ref[pl.ds(k * m_sub, m_sub), :], my))

        s = sx_ref[0] * sw_ref[0]
        w_waited = {1: False, 3: False, 2: False}
        w_cp_for_d = {1: w_cps[1], 3: w_cps[2], 2: w_cps[3]}
        for k in range(SUB):
            for d in (1, 3, 2):
                src = (my + N_DEV - d) % N_DEV
                with jax.named_scope(f"wait_recv#r={k}_d={d}"):
                    recv = pltpu.make_async_remote_copy(
                        src_ref=xv_ref.at[pl.ds(0, m_sub), :],
                        dst_ref=xg_ref.at[src, pl.ds(k * m_sub, m_sub), :],
                        send_sem=send_sems.at[d - 1, k],
                        recv_sem=recv_sems.at[d - 1, k],
                        device_id=(my,),
                        device_id_type=pl.DeviceIdType.MESH,
                    )
                    recv.wait_recv()
                    if not w_waited[d]:
                        w_cp_for_d[d].wait()
                        w_waited[d] = True
                with jax.named_scope(f"gemm#r={k}_d={d}"):
                    acc[k] = acc[k] + partial(
                        xg_ref[src, pl.ds(k * m_sub, m_sub), :], src)
            with jax.named_scope(f"epilogue#r={k}"):
                y = acc[k] * s
                out_ref[pl.ds(k * m_sub, m_sub), :] = y * jax.nn.sigmoid(y)

        with jax.named_scope("wait_send"):
            for rdma in sends:
                rdma.wait_send()

    x = pltpu.with_memory_space_constraint(x, pltpu.MemorySpace.HBM)
    w_mat = pltpu.with_memory_space_constraint(w_mat, pltpu.MemorySpace.HBM)
    return pl.pallas_call(
        body,
        out_shape=jax.ShapeDtypeStruct((m_per, n), jnp.float32),
        in_specs=[
            pl.BlockSpec(memory_space=pltpu.MemorySpace.HBM),
            pl.BlockSpec(memory_space=pltpu.MemorySpace.HBM),
            pl.BlockSpec(memory_space=pltpu.SMEM),
            pl.BlockSpec(memory_space=pltpu.SMEM),
        ],
        out_specs=pl.BlockSpec(memory_space=pltpu.VMEM),
        scratch_shapes=[
            pltpu.VMEM((N_DEV, m_per, k_per), jnp.int8),
            pltpu.VMEM((m_per, k_per), jnp.int8),
            pltpu.VMEM((N_DEV, k_per, n), jnp.int8),
            pltpu.SemaphoreType.DMA((N_DEV - 1, SUB)),
            pltpu.SemaphoreType.DMA((N_DEV - 1, SUB)),
            pltpu.SemaphoreType.DMA,
            pltpu.SemaphoreType.DMA((N_DEV,)),
        ],
        compiler_params=pltpu.CompilerParams(collective_id=0),
    )(x, w_mat, scale_x, scale_w)


# device time: 33190 ns/iter; 1.2374x vs baseline; 1.0176x over previous
import jax
import jax.numpy as jnp
from jax import lax
from jax.experimental import pallas as pl
from jax.experimental.pallas import tpu as pltpu

N_DEV = 4
SUB = 8


def kernel(x, w_mat, scale_x, scale_w):
    m_tot, k_per = x.shape
    _, n = w_mat.shape
    m_per = m_tot // N_DEV
    m_sub = m_per // SUB

    def body(x_ref, w_ref, sx_ref, sw_ref, out_ref, xg_ref, xv_ref, wv_ref,
             send_sems, recv_sems, x_sem, w_sems):
        my = lax.axis_index("i")

        with jax.named_scope("stage"):
            x_cp = pltpu.make_async_copy(
                x_ref.at[pl.ds(my * m_per, m_per), :], xv_ref, x_sem)
            x_cp.start()
            w_cps = []
            for i, off in enumerate((0, 3, 1, 2)):
                j = (my + off) % N_DEV
                cp = pltpu.make_async_copy(
                    w_ref.at[pl.ds(j * k_per, k_per), :],
                    wv_ref.at[j], w_sems.at[i])
                cp.start()
                w_cps.append(cp)

        with jax.named_scope("barrier"):
            barrier_sem = pltpu.get_barrier_semaphore()
            for d in range(1, N_DEV):
                pl.semaphore_signal(
                    barrier_sem, inc=1,
                    device_id=((my + d) % N_DEV,),
                    device_id_type=pl.DeviceIdType.MESH,
                )
            pl.semaphore_wait(barrier_sem, N_DEV - 1)

        sends = []
        with jax.named_scope("send_issue"):
            for k in range(SUB):
                for d in (1, 3, 2):
                    peer = (my + d) % N_DEV
                    rdma = pltpu.make_async_remote_copy(
                        src_ref=x_ref.at[
                            pl.ds(peer * m_per + k * m_sub, m_sub), :],
                        dst_ref=xg_ref.at[my, pl.ds(k * m_sub, m_sub), :],
                        send_sem=send_sems.at[d - 1, k],
                        recv_sem=recv_sems.at[d - 1, k],
                        device_id=(peer,),
                        device_id_type=pl.DeviceIdType.MESH,
                    )
                    rdma.start()
                    sends.append(rdma)

        def partial(xs, k_slice):
            return lax.dot_general(
                xs.astype(jnp.bfloat16),
                wv_ref[k_slice].astype(jnp.bfloat16),
                (((1,), (0,)), ((), ())),
                preferred_element_type=jnp.float32,
            )

        acc = []
        with jax.named_scope("gemm_local"):
            x_cp.wait()
            w_cps[0].wait()
            for k in range(SUB):
                acc.append(
                    partial(xv_ref[pl.ds(k * m_sub, m_sub), :], my))

        s = sx_ref[0] * sw_ref[0]
        w_waited = {1: False, 3: False, 2: False}
        w_cp_for_d = {1: w_cps[1], 3: w_cps[2], 2: w_cps[3]}
        for k in range(SUB):
            for d in (1, 3, 2):
                src = (my + N_DEV - d) % N_DEV
                with jax.named_scope(f"wait_recv#r={k}_d={d}"):
                    recv = pltpu.make_async_remote_copy(
                        src_ref=xv_ref.at[pl.ds(0, m_sub), :],
                        dst_ref=xg_ref.at[src, pl.ds(k * m_sub, m_sub), :],
                        send_sem=send_sems.at[d - 1, k],
                        recv_sem=recv_sems.at[d - 1, k],
                        device_id=(my,),
                        device_id_type=pl.DeviceIdType.MESH,
                    )
                    recv.wait_recv()
                    if not w_waited[d]:
                        w_cp_for_d[d].wait()
                        w_waited[d] = True
                with jax.named_scope(f"gemm#r={k}_d={d}"):
                    acc[k] = acc[k] + partial(
                        xg_ref[src, pl.ds(k * m_sub, m_sub), :], src)
            with jax.named_scope(f"epilogue#r={k}"):
                y = acc[k] * s
                out_ref[pl.ds(k * m_sub, m_sub), :] = y * jax.nn.sigmoid(y)

        with jax.named_scope("wait_send"):
            for rdma in sends:
                rdma.wait_send()

    x = pltpu.with_memory_space_constraint(x, pltpu.MemorySpace.HBM)
    w_mat = pltpu.with_memory_space_constraint(w_mat, pltpu.MemorySpace.HBM)
    return pl.pallas_call(
        body,
        out_shape=jax.ShapeDtypeStruct((m_per, n), jnp.float32),
        in_specs=[
            pl.BlockSpec(memory_space=pltpu.MemorySpace.HBM),
            pl.BlockSpec(memory_space=pltpu.MemorySpace.HBM),
            pl.BlockSpec(memory_space=pltpu.SMEM),
            pl.BlockSpec(memory_space=pltpu.SMEM),
        ],
        out_specs=pl.BlockSpec(memory_space=pltpu.VMEM),
        scratch_shapes=[
            pltpu.VMEM((N_DEV, m_per, k_per), jnp.int8),
            pltpu.VMEM((m_per, k_per), jnp.int8),
            pltpu.VMEM((N_DEV, k_per, n), jnp.int8),
            pltpu.SemaphoreType.DMA((N_DEV - 1, SUB)),
            pltpu.SemaphoreType.DMA((N_DEV - 1, SUB)),
            pltpu.SemaphoreType.DMA,
            pltpu.SemaphoreType.DMA((N_DEV,)),
        ],
        compiler_params=pltpu.CompilerParams(collective_id=0),
    )(x, w_mat, scale_x, scale_w)
